# baseline (speedup 1.0000x reference)
import jax
import jax.numpy as jnp
from jax.experimental import pallas as pl
from jax.experimental.pallas import tpu as pltpu


def _gcn_fused_kernel(x_ref, adj_ref, w1_ref, w2_ref, w3_ref, out_ref):
    x = x_ref[0]
    adj = adj_ref[0]

    def layer(h, w):
        support = jnp.dot(h, w, preferred_element_type=jnp.float32)
        agg = jnp.dot(adj, support, preferred_element_type=jnp.float32)
        return jnp.maximum(agg, 0.0)

    h = layer(x, w1_ref[...])
    h = layer(h, w2_ref[...])
    h = layer(h, w3_ref[...])

    m = jnp.max(h, axis=0, keepdims=True)
    shifted = h - m
    lse = jnp.log(jnp.sum(jnp.exp(shifted), axis=0, keepdims=True))
    out_ref[0] = shifted - lse


def kernel(x, adj, W1, W2, W3):
    B, N, D = x.shape
    L = W3.shape[1]
    return pl.pallas_call(
        _gcn_fused_kernel,
        grid=(B,),
        in_specs=[
            pl.BlockSpec((1, N, D), lambda b: (b, 0, 0)),
            pl.BlockSpec((1, N, N), lambda b: (b, 0, 0)),
            pl.BlockSpec((D, D), lambda b: (0, 0)),
            pl.BlockSpec((D, D), lambda b: (0, 0)),
            pl.BlockSpec((D, L), lambda b: (0, 0)),
        ],
        out_specs=pl.BlockSpec((1, N, L), lambda b: (b, 0, 0)),
        out_shape=jax.ShapeDtypeStruct((B, N, L), jnp.float32),
        compiler_params=pltpu.CompilerParams(
            dimension_semantics=("parallel",),
        ),
    )(x, adj, W1, W2, W3)


# speedup vs baseline: 1.0118x; 1.0118x over previous
import jax
import jax.numpy as jnp
from jax.experimental import pallas as pl
from jax.experimental.pallas import tpu as pltpu


def _gcn_fused_kernel(x_ref, adj_ref, w1_ref, w2_ref, w3_ref, out_ref):
    x = x_ref[0].astype(jnp.bfloat16)
    adj = adj_ref[0].astype(jnp.bfloat16)

    def layer(h, w):
        support = jnp.dot(h, w, preferred_element_type=jnp.float32)
        agg = jnp.dot(adj, support.astype(jnp.bfloat16),
                      preferred_element_type=jnp.float32)
        return jnp.maximum(agg, 0.0)

    h = layer(x, w1_ref[...].astype(jnp.bfloat16))
    h = layer(h.astype(jnp.bfloat16), w2_ref[...].astype(jnp.bfloat16))
    h = layer(h.astype(jnp.bfloat16), w3_ref[...].astype(jnp.bfloat16))

    m = jnp.max(h, axis=0, keepdims=True)
    shifted = h - m
    lse = jnp.log(jnp.sum(jnp.exp(shifted), axis=0, keepdims=True))
    out_ref[0] = shifted - lse


def kernel(x, adj, W1, W2, W3):
    B, N, D = x.shape
    L = W3.shape[1]
    return pl.pallas_call(
        _gcn_fused_kernel,
        grid=(B,),
        in_specs=[
            pl.BlockSpec((1, N, D), lambda b: (b, 0, 0)),
            pl.BlockSpec((1, N, N), lambda b: (b, 0, 0)),
            pl.BlockSpec((D, D), lambda b: (0, 0)),
            pl.BlockSpec((D, D), lambda b: (0, 0)),
            pl.BlockSpec((D, L), lambda b: (0, 0)),
        ],
        out_specs=pl.BlockSpec((1, N, L), lambda b: (b, 0, 0)),
        out_shape=jax.ShapeDtypeStruct((B, N, L), jnp.float32),
        compiler_params=pltpu.CompilerParams(
            dimension_semantics=("parallel",),
        ),
    )(x, adj, W1, W2, W3)


# speedup vs baseline: 1.4632x; 1.4461x over previous
import jax
import jax.numpy as jnp
from jax.experimental import pallas as pl
from jax.experimental.pallas import tpu as pltpu


def _gcn_fused_kernel(x_ref, adj_ref, w1_ref, w2_ref, w3_ref, out_ref):
    x = x_ref[0]
    adj = adj_ref[0]

    S = 256

    def layer(h, w):
        support = jnp.dot(h, w, preferred_element_type=jnp.float32)
        slabs = [
            jnp.maximum(
                jnp.dot(adj[m * S:(m + 1) * S, :], support,
                        preferred_element_type=jnp.float32), 0.0)
            for m in range(adj.shape[0] // S)
        ]
        return jnp.concatenate(slabs, axis=0)

    h = layer(x, w1_ref[...])
    h = layer(h, w2_ref[...])
    h = layer(h, w3_ref[...])

    m = jnp.max(h, axis=0, keepdims=True)
    shifted = h - m
    lse = jnp.log(jnp.sum(jnp.exp(shifted), axis=0, keepdims=True))
    out_ref[0] = shifted - lse


def kernel(x, adj, W1, W2, W3):
    B, N, D = x.shape
    L = W3.shape[1]
    return pl.pallas_call(
        _gcn_fused_kernel,
        grid=(B,),
        in_specs=[
            pl.BlockSpec((1, N, D), lambda b: (b, 0, 0)),
            pl.BlockSpec((1, N, N), lambda b: (b, 0, 0)),
            pl.BlockSpec((D, D), lambda b: (0, 0)),
            pl.BlockSpec((D, D), lambda b: (0, 0)),
            pl.BlockSpec((D, L), lambda b: (0, 0)),
        ],
        out_specs=pl.BlockSpec((1, N, L), lambda b: (b, 0, 0)),
        out_shape=jax.ShapeDtypeStruct((B, N, L), jnp.float32),
        compiler_params=pltpu.CompilerParams(
            dimension_semantics=("parallel",),
        ),
    )(x, adj, W1, W2, W3)


# speedup vs baseline: 1.5114x; 1.0330x over previous
import jax
import jax.numpy as jnp
from jax.experimental import pallas as pl
from jax.experimental.pallas import tpu as pltpu


def _gcn_fused_kernel(x_ref, adj_ref, w1_ref, w2_ref, w3_ref, out_ref):
    x = x_ref[0]
    adj = adj_ref[0]

    S = 256

    def layer(h, w):
        support = jnp.dot(h, w, preferred_element_type=jnp.float32)
        slabs = [
            jnp.maximum(
                jnp.dot(adj[m * S:(m + 1) * S, :], support,
                        preferred_element_type=jnp.float32), 0.0)
            for m in range(adj.shape[0] // S)
        ]
        return jnp.concatenate(slabs, axis=0)

    h = layer(x, w1_ref[...])
    h = layer(h, w2_ref[...])

    support = jnp.dot(h, w3_ref[...], preferred_element_type=jnp.float32)
    L = support.shape[1]
    m_run = jnp.zeros((1, L), jnp.float32)
    s_run = jnp.zeros((1, L), jnp.float32)
    h3_slabs = []
    for i in range(adj.shape[0] // S):
        hs = jnp.maximum(
            jnp.dot(adj[i * S:(i + 1) * S, :], support,
                    preferred_element_type=jnp.float32), 0.0)
        h3_slabs.append(hs)
        m_new = jnp.maximum(m_run, jnp.max(hs, axis=0, keepdims=True))
        s_run = (s_run * jnp.exp(m_run - m_new)
                 + jnp.sum(jnp.exp(hs - m_new), axis=0, keepdims=True))
        m_run = m_new

    lse = m_run + jnp.log(s_run)
    out_ref[0] = jnp.concatenate(h3_slabs, axis=0) - lse


def kernel(x, adj, W1, W2, W3):
    B, N, D = x.shape
    L = W3.shape[1]
    return pl.pallas_call(
        _gcn_fused_kernel,
        grid=(B,),
        in_specs=[
            pl.BlockSpec((1, N, D), lambda b: (b, 0, 0)),
            pl.BlockSpec((1, N, N), lambda b: (b, 0, 0)),
            pl.BlockSpec((D, D), lambda b: (0, 0)),
            pl.BlockSpec((D, D), lambda b: (0, 0)),
            pl.BlockSpec((D, L), lambda b: (0, 0)),
        ],
        out_specs=pl.BlockSpec((1, N, L), lambda b: (b, 0, 0)),
        out_shape=jax.ShapeDtypeStruct((B, N, L), jnp.float32),
        compiler_params=pltpu.CompilerParams(
            dimension_semantics=("parallel",),
        ),
    )(x, adj, W1, W2, W3)


# speedup vs baseline: 1.5157x; 1.0028x over previous
import jax
import jax.numpy as jnp
from jax.experimental import pallas as pl
from jax.experimental.pallas import tpu as pltpu


def _gcn_fused_kernel(x_ref, adj_ref, w1_ref, w2_ref, w3_ref, out_ref):
    x = x_ref[0]
    adj = adj_ref[0]

    S = 256

    def layer(h, w):
        support = jnp.dot(h, w, preferred_element_type=jnp.float32)
        slabs = [
            jnp.maximum(
                jnp.dot(adj[m * S:(m + 1) * S, :], support,
                        preferred_element_type=jnp.float32), 0.0
            )
            for m in range(adj.shape[0] // S)
        ]
        return jnp.concatenate(slabs, axis=0)

    h = layer(x, w1_ref[...])
    h = layer(h, w2_ref[...])

    support = jnp.dot(h, w3_ref[...], preferred_element_type=jnp.float32)
    L = support.shape[1]
    m_run = jnp.zeros((1, L), jnp.float32)
    s_run = jnp.zeros((1, L), jnp.float32)
    S3 = 256
    h3_slabs = []
    for i in range(adj.shape[0] // S3):
        hs = jnp.maximum(
            jnp.dot(adj[i * S3:(i + 1) * S3, :], support,
                    preferred_element_type=jnp.float32), 0.0)
        h3_slabs.append(hs)
        m_new = jnp.maximum(m_run, jnp.max(hs, axis=0, keepdims=True))
        s_run = (s_run * jnp.exp(m_run - m_new)
                 + jnp.sum(jnp.exp(hs - m_new), axis=0, keepdims=True))
        m_run = m_new

    lse = m_run + jnp.log(s_run)
    out_ref[0] = jnp.concatenate(h3_slabs, axis=0) - lse


def kernel(x, adj, W1, W2, W3):
    B, N, D = x.shape
    L = W3.shape[1]
    return pl.pallas_call(
        _gcn_fused_kernel,
        grid=(B,),
        in_specs=[
            pl.BlockSpec((1, N, D), lambda b: (b, 0, 0)),
            pl.BlockSpec((1, N, N), lambda b: (b, 0, 0)),
            pl.BlockSpec((D, D), lambda b: (0, 0)),
            pl.BlockSpec((D, D), lambda b: (0, 0)),
            pl.BlockSpec((D, L), lambda b: (0, 0)),
        ],
        out_specs=pl.BlockSpec((1, N, L), lambda b: (b, 0, 0)),
        out_shape=jax.ShapeDtypeStruct((B, N, L), jnp.float32),
        compiler_params=pltpu.CompilerParams(
            dimension_semantics=("parallel",),
        ),
    )(x, adj, W1, W2, W3)
